# baseline (speedup 1.0000x reference)
import functools

import jax
import jax.numpy as jnp
from jax import lax
from jax.experimental import pallas as pl
from jax.experimental.pallas import tpu as pltpu
from jax.experimental.pallas import tpu_sc as plsc

N_ROWS = 1000
GRID1 = 17
LATENT = 16
B = 1024
P = GRID1 * GRID1 * GRID1
L = LATENT

NW = 32
PPW = 156
NIN = 4
NOUT = 3
STEP = 12
NCORES = 2


def _body(idx_hbm, table_hbm, out_hbm, idx_v, in_v, out_v, *sems):
    wid = lax.axis_index("s") * NCORES + lax.axis_index("c")
    start = jnp.minimum(wid * 154, P - PPW)
    isems = sems[:NIN]
    osems = sems[NIN:]

    pltpu.sync_copy(idx_hbm, idx_v)

    def in_copy(i, s):
        return pltpu.make_async_copy(table_hbm.at[start + i], in_v.at[s], isems[s])

    def out_copy(i, s):
        return pltpu.make_async_copy(out_v.at[s], out_hbm.at[start + i], osems[s])

    for s in range(NIN):
        in_copy(s, s).start()

    def gather_plane(si, so):
        src = in_v.at[si]
        dst = out_v.at[so]

        @plsc.parallel_loop(0, B // 16, unroll=4)
        def g_body(g):
            c = idx_v[pl.ds(16 * g, 16)]
            for l in range(L):
                row = jnp.full((16,), l, jnp.int32)
                dst[l, pl.ds(16 * g, 16)] = plsc.load_gather(src, [row, c])

    def outer(o, carry):
        for t in range(STEP):
            i = STEP * o + t
            si, so = t % NIN, t % NOUT

            if t >= NOUT:
                out_copy(i - NOUT, so).wait()
            else:

                @pl.when(o >= 1)
                def _():
                    out_copy(i - NOUT, so).wait()

            in_copy(i, si).wait()
            gather_plane(si, so)
            out_copy(i, so).start()

            if t < STEP - NIN:
                in_copy(i + NIN, si).start()
            else:

                @pl.when(o < PPW // STEP - 1)
                def _():
                    in_copy(i + NIN, si).start()

        return carry

    lax.fori_loop(0, PPW // STEP, outer, 0)

    for k in range(NOUT):
        out_copy(PPW - NOUT + k, (PPW - NOUT + k) % NOUT).wait()


def kernel(indices, embeddings):
    table = jnp.transpose(embeddings, (1, 2, 3, 4, 0)).reshape(P, L, N_ROWS)
    idx32 = indices.astype(jnp.int32)

    k = functools.partial(
        pl.kernel,
        mesh=plsc.VectorSubcoreMesh(core_axis_name="c", subcore_axis_name="s"),
        out_type=jax.ShapeDtypeStruct((P, L, B), jnp.float32),
        scratch_types=[
            pltpu.VMEM((B,), jnp.int32),
            pltpu.VMEM((NIN, L, N_ROWS), jnp.float32),
            pltpu.VMEM((NOUT, L, B), jnp.float32),
        ]
        + [pltpu.SemaphoreType.DMA] * (NIN + NOUT),
        compiler_params=pltpu.CompilerParams(needs_layout_passes=False),
    )(_body)

    out = k(idx32, table)
    return jnp.transpose(
        out.reshape(GRID1, GRID1, GRID1, LATENT, B), (4, 0, 1, 2, 3)
    )


# speedup vs baseline: 1.0237x; 1.0237x over previous
import functools

import jax
import jax.numpy as jnp
from jax import lax
from jax.experimental import pallas as pl
from jax.experimental.pallas import tpu as pltpu
from jax.experimental.pallas import tpu_sc as plsc

N_ROWS = 1000
GRID1 = 17
LATENT = 16
B = 1024
P = GRID1 * GRID1 * GRID1
L = LATENT

NW = 32
PPW = 156
NSLOTS = 3
NCORES = 2


def _body(idx_hbm, table_hbm, out_hbm, idx_v, in_v, out_v, *sems):
    wid = lax.axis_index("s") * NCORES + lax.axis_index("c")
    start = jnp.minimum(wid * 154, P - PPW)
    isems = sems[:NSLOTS]
    osems = sems[NSLOTS:]

    pltpu.sync_copy(idx_hbm, idx_v)

    def in_copy(i, s):
        return pltpu.make_async_copy(table_hbm.at[start + i], in_v.at[s], isems[s])

    def out_copy(i, s):
        return pltpu.make_async_copy(out_v.at[s], out_hbm.at[start + i], osems[s])

    for s in range(NSLOTS):
        in_copy(s, s).start()

    def gather_plane(s):
        src = in_v.at[s]
        dst = out_v.at[s]

        @plsc.parallel_loop(0, B // 16, unroll=4)
        def g_body(g):
            c = idx_v[pl.ds(16 * g, 16)]
            for l in range(L):
                row = jnp.full((16,), l, jnp.int32)
                dst[l, pl.ds(16 * g, 16)] = plsc.load_gather(src, [row, c])

    def outer(o, carry):
        for s in range(NSLOTS):
            i = NSLOTS * o + s

            @pl.when(o >= 1)
            def _():
                out_copy(i - NSLOTS, s).wait()

            in_copy(i, s).wait()
            gather_plane(s)
            out_copy(i, s).start()

            @pl.when(o < PPW // NSLOTS - 1)
            def _():
                in_copy(i + NSLOTS, s).start()

        return carry

    lax.fori_loop(0, PPW // NSLOTS, outer, 0)

    for s in range(NSLOTS):
        out_copy(PPW - NSLOTS + s, s).wait()


def kernel(indices, embeddings):
    table = jnp.transpose(embeddings, (1, 2, 3, 4, 0)).reshape(P, L, N_ROWS)
    idx32 = indices.astype(jnp.int32)

    k = functools.partial(
        pl.kernel,
        mesh=plsc.VectorSubcoreMesh(core_axis_name="c", subcore_axis_name="s"),
        out_type=jax.ShapeDtypeStruct((P, L, B), jnp.float32),
        scratch_types=[
            pltpu.VMEM((B,), jnp.int32),
            pltpu.VMEM((NSLOTS, L, N_ROWS), jnp.float32),
            pltpu.VMEM((NSLOTS, L, B), jnp.float32),
        ]
        + [pltpu.SemaphoreType.DMA] * (2 * NSLOTS),
        compiler_params=pltpu.CompilerParams(needs_layout_passes=False),
    )(_body)

    out = k(idx32, table)
    return jnp.transpose(
        out.reshape(GRID1, GRID1, GRID1, LATENT, B), (4, 0, 1, 2, 3)
    )
